# Initial kernel scaffold; baseline (speedup 1.0000x reference)
import jax
import jax.numpy as jnp
from jax.experimental import pallas as pl


def kernel(embedded, symbol):
    raise NotImplementedError("write your pallas kernel here")


# speedup vs baseline: 7.6993x; 7.6993x over previous
import math

import numpy as np
import jax
import jax.numpy as jnp
from jax.experimental import pallas as pl

D_MODEL = 1024
MAX_LENGTH = 8192
_PAD = 0
_LB = 1024


def _pe_table():
    position = np.arange(MAX_LENGTH, dtype=np.float32)[:, None]
    scale = -math.log(10000.0) / D_MODEL
    div = np.exp(np.arange(0, D_MODEL, 2, dtype=np.float32) * scale)
    pe = np.zeros((MAX_LENGTH, D_MODEL), dtype=np.float32)
    pe[:, 0::2] = np.sin(position * div)
    pe[:, 1::2] = np.cos(position * div)
    return pe


_PE = _pe_table()


def _body(sym_ref, emb_ref, pe_ref, out_ref):
    mask = (sym_ref[0] != _PAD).astype(jnp.float32)
    out_ref[0] = emb_ref[0] + pe_ref[...] * mask


def kernel(embedded, symbol):
    B, L = symbol.shape
    nl = L // _LB
    sym3 = symbol.reshape(B, L, 1)
    pe = jnp.asarray(_PE)
    return pl.pallas_call(
        _body,
        grid=(nl, B),
        in_specs=[
            pl.BlockSpec((1, _LB, 1), lambda i, b: (b, i, 0)),
            pl.BlockSpec((1, _LB, D_MODEL), lambda i, b: (b, i, 0)),
            pl.BlockSpec((_LB, D_MODEL), lambda i, b: (i, 0)),
        ],
        out_specs=pl.BlockSpec((1, _LB, D_MODEL), lambda i, b: (b, i, 0)),
        out_shape=jax.ShapeDtypeStruct((B, L, D_MODEL), jnp.float32),
    )(sym3, embedded, pe)
